# initial kernel scaffold (unmeasured)
import jax
import jax.numpy as jnp
from jax import lax
from jax.experimental import pallas as pl
from jax.experimental.pallas import tpu as pltpu


def kernel(
    x,
):
    def body(*refs):
        pass

    out_shape = jax.ShapeDtypeStruct(..., jnp.float32)
    return pl.pallas_call(body, out_shape=out_shape)(...)


# baseline (device time: 450052 ns/iter reference)
import jax
import jax.numpy as jnp
from jax import lax
from jax.experimental import pallas as pl
from jax.experimental.pallas import tpu as pltpu

N_DEV = 16


def kernel(x):
    m_per, n = x.shape
    chunk = m_per // N_DEV

    def body(x_ref, out_ref, buf_ref, rs_send_sems, rs_recv_sems,
             ag_send_sems, ag_recv_sems, credit_sem):
        my = lax.axis_index("i")
        left = lax.rem(my + N_DEV - 1, N_DEV)
        right = lax.rem(my + 1, N_DEV)

        def cidx(k):
            return lax.rem(my + k + 2 * N_DEV, N_DEV)

        def csl(idx):
            return pl.ds(idx * chunk, chunk)

        barrier_sem = pltpu.get_barrier_semaphore()
        pl.semaphore_signal(barrier_sem, 1, device_id=(left,),
                            device_id_type=pl.DeviceIdType.MESH)
        pl.semaphore_signal(barrier_sem, 1, device_id=(right,),
                            device_id_type=pl.DeviceIdType.MESH)
        pl.semaphore_wait(barrier_sem, 2)

        for s in range(N_DEV - 1):
            r = (s + 1) % 2
            if s >= 2:
                pl.semaphore_wait(credit_sem, 1)
            if s == 0:
                src = x_ref.at[csl(cidx(-1))]
            else:
                src = buf_ref.at[s % 2]
            rdma = pltpu.make_async_remote_copy(
                src_ref=src,
                dst_ref=buf_ref.at[r],
                send_sem=rs_send_sems.at[s % 2],
                recv_sem=rs_recv_sems.at[r],
                device_id=(right,),
                device_id_type=pl.DeviceIdType.MESH,
            )
            rdma.start()
            rdma.wait()
            buf_ref[r] = buf_ref[r] + x_ref[csl(cidx(-2 - s)), :]
            if 1 <= s <= N_DEV - 3:
                pl.semaphore_signal(credit_sem, 1, device_id=(left,),
                                    device_id_type=pl.DeviceIdType.MESH)

        out_ref[csl(my), :] = buf_ref[1]

        for h in range(N_DEV - 1):
            if h >= 2:
                pl.semaphore_wait(credit_sem, 1)
            c_send = cidx(-h)
            rdma = pltpu.make_async_remote_copy(
                src_ref=out_ref.at[csl(c_send)],
                dst_ref=out_ref.at[csl(c_send)],
                send_sem=ag_send_sems.at[h % 2],
                recv_sem=ag_recv_sems.at[h % 2],
                device_id=(right,),
                device_id_type=pl.DeviceIdType.MESH,
            )
            rdma.start()
            rdma.wait()
            if h <= N_DEV - 4:
                pl.semaphore_signal(credit_sem, 1, device_id=(left,),
                                    device_id_type=pl.DeviceIdType.MESH)

    return pl.pallas_call(
        body,
        out_shape=jax.ShapeDtypeStruct((m_per, n), x.dtype),
        in_specs=[pl.BlockSpec(memory_space=pltpu.VMEM)],
        out_specs=pl.BlockSpec(memory_space=pltpu.VMEM),
        scratch_shapes=[
            pltpu.VMEM((2, chunk, n), x.dtype),
            pltpu.SemaphoreType.DMA((2,)),
            pltpu.SemaphoreType.DMA((2,)),
            pltpu.SemaphoreType.DMA((2,)),
            pltpu.SemaphoreType.DMA((2,)),
            pltpu.SemaphoreType.REGULAR,
        ],
        compiler_params=pltpu.CompilerParams(collective_id=0),
    )(x)


# device time: 242749 ns/iter; 1.8540x vs baseline; 1.8540x over previous
import jax
import jax.numpy as jnp
from jax import lax
from jax.experimental import pallas as pl
from jax.experimental.pallas import tpu as pltpu

N_DEV = 16


def kernel(x):
    m_per, n = x.shape
    chunk = m_per // N_DEV
    half = chunk // 2

    def body(x_ref, out_ref, cwbuf, ccwbuf,
             cw_rs_send, cw_rs_recv, ccw_rs_send, ccw_rs_recv,
             cw_ag_send, cw_ag_recv, ccw_ag_send, ccw_ag_recv):
        my = lax.axis_index("i")
        left = lax.rem(my + N_DEV - 1, N_DEV)
        right = lax.rem(my + 1, N_DEV)

        def cidx(k):
            return lax.rem(my + k + 2 * N_DEV, N_DEV)

        def top(idx):
            return pl.ds(idx * chunk, half)

        def bot(idx):
            return pl.ds(idx * chunk + half, half)

        barrier_sem = pltpu.get_barrier_semaphore()
        pl.semaphore_signal(barrier_sem, 1, device_id=(left,),
                            device_id_type=pl.DeviceIdType.MESH)
        pl.semaphore_signal(barrier_sem, 1, device_id=(right,),
                            device_id_type=pl.DeviceIdType.MESH)
        pl.semaphore_wait(barrier_sem, 2)

        def send_cw_rs(s, src):
            d = pltpu.make_async_remote_copy(
                src_ref=src, dst_ref=cwbuf.at[s],
                send_sem=cw_rs_send.at[s % 2], recv_sem=cw_rs_recv.at[s],
                device_id=(right,), device_id_type=pl.DeviceIdType.MESH)
            d.start()
            return d

        def send_ccw_rs(s, src):
            d = pltpu.make_async_remote_copy(
                src_ref=src, dst_ref=ccwbuf.at[s],
                send_sem=ccw_rs_send.at[s % 2], recv_sem=ccw_rs_recv.at[s],
                device_id=(left,), device_id_type=pl.DeviceIdType.MESH)
            d.start()
            return d

        cw_descs = [send_cw_rs(0, x_ref.at[top(cidx(-1))])]
        ccw_descs = [send_ccw_rs(0, x_ref.at[bot(cidx(+1))])]
        for s in range(N_DEV - 1):
            cw_descs[s].wait_recv()
            cwbuf[s] = cwbuf[s] + x_ref[top(cidx(-2 - s)), :]
            if s < N_DEV - 2:
                if s >= 1:
                    cw_descs[s - 1].wait_send()
                cw_descs.append(send_cw_rs(s + 1, cwbuf.at[s]))
            ccw_descs[s].wait_recv()
            ccwbuf[s] = ccwbuf[s] + x_ref[bot(cidx(+2 + s)), :]
            if s < N_DEV - 2:
                if s >= 1:
                    ccw_descs[s - 1].wait_send()
                ccw_descs.append(send_ccw_rs(s + 1, ccwbuf.at[s]))

        out_ref[top(my), :] = cwbuf[N_DEV - 2]
        out_ref[bot(my), :] = ccwbuf[N_DEV - 2]

        for d in (cw_descs[-2], cw_descs[-1], ccw_descs[-2], ccw_descs[-1]):
            d.wait_send()

        def send_cw_ag(h):
            c = cidx(-h)
            d = pltpu.make_async_remote_copy(
                src_ref=out_ref.at[top(c)], dst_ref=out_ref.at[top(c)],
                send_sem=cw_ag_send.at[h % 2], recv_sem=cw_ag_recv.at[h],
                device_id=(right,), device_id_type=pl.DeviceIdType.MESH)
            d.start()
            return d

        def send_ccw_ag(h):
            c = cidx(+h)
            d = pltpu.make_async_remote_copy(
                src_ref=out_ref.at[bot(c)], dst_ref=out_ref.at[bot(c)],
                send_sem=ccw_ag_send.at[h % 2], recv_sem=ccw_ag_recv.at[h],
                device_id=(left,), device_id_type=pl.DeviceIdType.MESH)
            d.start()
            return d

        cw_ag = [send_cw_ag(0)]
        ccw_ag = [send_ccw_ag(0)]
        for h in range(N_DEV - 1):
            cw_ag[h].wait_recv()
            if h < N_DEV - 2:
                if h >= 1:
                    cw_ag[h - 1].wait_send()
                cw_ag.append(send_cw_ag(h + 1))
            ccw_ag[h].wait_recv()
            if h < N_DEV - 2:
                if h >= 1:
                    ccw_ag[h - 1].wait_send()
                ccw_ag.append(send_ccw_ag(h + 1))

        for d in (cw_ag[-2], cw_ag[-1], ccw_ag[-2], ccw_ag[-1]):
            d.wait_send()

    nslots = N_DEV - 1
    return pl.pallas_call(
        body,
        out_shape=jax.ShapeDtypeStruct((m_per, n), x.dtype),
        in_specs=[pl.BlockSpec(memory_space=pltpu.VMEM)],
        out_specs=pl.BlockSpec(memory_space=pltpu.VMEM),
        scratch_shapes=[
            pltpu.VMEM((nslots, half, n), x.dtype),
            pltpu.VMEM((nslots, half, n), x.dtype),
            pltpu.SemaphoreType.DMA((2,)),
            pltpu.SemaphoreType.DMA((nslots,)),
            pltpu.SemaphoreType.DMA((2,)),
            pltpu.SemaphoreType.DMA((nslots,)),
            pltpu.SemaphoreType.DMA((2,)),
            pltpu.SemaphoreType.DMA((nslots,)),
            pltpu.SemaphoreType.DMA((2,)),
            pltpu.SemaphoreType.DMA((nslots,)),
        ],
        compiler_params=pltpu.CompilerParams(collective_id=0),
    )(x)


# device time: 191039 ns/iter; 2.3558x vs baseline; 1.2707x over previous
import jax
import jax.numpy as jnp
from jax import lax
from jax.experimental import pallas as pl
from jax.experimental.pallas import tpu as pltpu

N_DEV = 16
NSUB = 2


def kernel(x):
    m_per, n = x.shape
    chunk = m_per // N_DEV
    half = chunk // 2
    sub = half // NSUB
    nst = N_DEV - 1

    def body(x_ref, out_ref, cwbuf, ccwbuf,
             cw_rs_send, cw_rs_recv, ccw_rs_send, ccw_rs_recv,
             cw_ag_send, cw_ag_recv, ccw_ag_send, ccw_ag_recv):
        my = lax.axis_index("i")
        left = lax.rem(my + N_DEV - 1, N_DEV)
        right = lax.rem(my + 1, N_DEV)

        def cidx(k):
            return lax.rem(my + k + 2 * N_DEV, N_DEV)

        def topsub(idx, b):
            return pl.ds(idx * chunk + b * sub, sub)

        def botsub(idx, b):
            return pl.ds(idx * chunk + half + b * sub, sub)

        barrier_sem = pltpu.get_barrier_semaphore()
        pl.semaphore_signal(barrier_sem, 1, device_id=(left,),
                            device_id_type=pl.DeviceIdType.MESH)
        pl.semaphore_signal(barrier_sem, 1, device_id=(right,),
                            device_id_type=pl.DeviceIdType.MESH)
        pl.semaphore_wait(barrier_sem, 2)

        def rs_send(s, b, cw):
            if cw:
                src = (x_ref.at[topsub(cidx(-1), b)] if s == 0
                       else cwbuf.at[s - 1, pl.ds(b * sub, sub)])
                d = pltpu.make_async_remote_copy(
                    src_ref=src,
                    dst_ref=cwbuf.at[s, pl.ds(b * sub, sub)],
                    send_sem=cw_rs_send.at[(2 * s + b) % 4],
                    recv_sem=cw_rs_recv.at[s, b],
                    device_id=(right,), device_id_type=pl.DeviceIdType.MESH)
            else:
                src = (x_ref.at[botsub(cidx(+1), b)] if s == 0
                       else ccwbuf.at[s - 1, pl.ds(b * sub, sub)])
                d = pltpu.make_async_remote_copy(
                    src_ref=src,
                    dst_ref=ccwbuf.at[s, pl.ds(b * sub, sub)],
                    send_sem=ccw_rs_send.at[(2 * s + b) % 4],
                    recv_sem=ccw_rs_recv.at[s, b],
                    device_id=(left,), device_id_type=pl.DeviceIdType.MESH)
            d.start()
            return d

        cw_d = {(0, b): rs_send(0, b, True) for b in range(NSUB)}
        ccw_d = {(0, b): rs_send(0, b, False) for b in range(NSUB)}
        for s in range(nst):
            for b in range(NSUB):
                cw_d[(s, b)].wait_recv()
                cwbuf[s, pl.ds(b * sub, sub)] = (
                    cwbuf[s, pl.ds(b * sub, sub)]
                    + x_ref[topsub(cidx(-2 - s), b), :])
                if s < nst - 1:
                    if s >= 1:
                        cw_d[(s - 1, b)].wait_send()
                    cw_d[(s + 1, b)] = rs_send(s + 1, b, True)
                ccw_d[(s, b)].wait_recv()
                ccwbuf[s, pl.ds(b * sub, sub)] = (
                    ccwbuf[s, pl.ds(b * sub, sub)]
                    + x_ref[botsub(cidx(+2 + s), b), :])
                if s < nst - 1:
                    if s >= 1:
                        ccw_d[(s - 1, b)].wait_send()
                    ccw_d[(s + 1, b)] = rs_send(s + 1, b, False)

        def ag_send(h, b, cw):
            if cw:
                c = cidx(-h)
                src = (cwbuf.at[nst - 1, pl.ds(b * sub, sub)] if h == 0
                       else out_ref.at[topsub(c, b)])
                d = pltpu.make_async_remote_copy(
                    src_ref=src, dst_ref=out_ref.at[topsub(c, b)],
                    send_sem=cw_ag_send.at[(2 * h + b) % 4],
                    recv_sem=cw_ag_recv.at[h, b],
                    device_id=(right,), device_id_type=pl.DeviceIdType.MESH)
            else:
                c = cidx(+h)
                src = (ccwbuf.at[nst - 1, pl.ds(b * sub, sub)] if h == 0
                       else out_ref.at[botsub(c, b)])
                d = pltpu.make_async_remote_copy(
                    src_ref=src, dst_ref=out_ref.at[botsub(c, b)],
                    send_sem=ccw_ag_send.at[(2 * h + b) % 4],
                    recv_sem=ccw_ag_recv.at[h, b],
                    device_id=(left,), device_id_type=pl.DeviceIdType.MESH)
            d.start()
            return d

        cw_a = {(0, b): ag_send(0, b, True) for b in range(NSUB)}
        ccw_a = {(0, b): ag_send(0, b, False) for b in range(NSUB)}
        out_ref[pl.ds(my * chunk, half), :] = cwbuf[nst - 1]
        out_ref[pl.ds(my * chunk + half, half), :] = ccwbuf[nst - 1]

        for h in range(nst):
            for b in range(NSUB):
                cw_a[(h, b)].wait_recv()
                if h < nst - 1:
                    if h >= 1:
                        cw_a[(h - 1, b)].wait_send()
                    cw_a[(h + 1, b)] = ag_send(h + 1, b, True)
                ccw_a[(h, b)].wait_recv()
                if h < nst - 1:
                    if h >= 1:
                        ccw_a[(h - 1, b)].wait_send()
                    ccw_a[(h + 1, b)] = ag_send(h + 1, b, False)

        for dct in (cw_d, ccw_d, cw_a, ccw_a):
            for s in (nst - 2, nst - 1):
                for b in range(NSUB):
                    dct[(s, b)].wait_send()

    return pl.pallas_call(
        body,
        out_shape=jax.ShapeDtypeStruct((m_per, n), x.dtype),
        in_specs=[pl.BlockSpec(memory_space=pltpu.VMEM)],
        out_specs=pl.BlockSpec(memory_space=pltpu.VMEM),
        scratch_shapes=[
            pltpu.VMEM((nst, half, n), x.dtype),
            pltpu.VMEM((nst, half, n), x.dtype),
            pltpu.SemaphoreType.DMA((4,)),
            pltpu.SemaphoreType.DMA((nst, NSUB)),
            pltpu.SemaphoreType.DMA((4,)),
            pltpu.SemaphoreType.DMA((nst, NSUB)),
            pltpu.SemaphoreType.DMA((4,)),
            pltpu.SemaphoreType.DMA((nst, NSUB)),
            pltpu.SemaphoreType.DMA((4,)),
            pltpu.SemaphoreType.DMA((nst, NSUB)),
        ],
        compiler_params=pltpu.CompilerParams(collective_id=0),
    )(x)
